# baseline (speedup 1.0000x reference)
import functools

import jax
import jax.numpy as jnp
from jax import lax
from jax.experimental import pallas as pl
from jax.experimental.pallas import tpu as pltpu
from jax.experimental.pallas import tpu_sc as plsc

EMBED_DIM = 64
ROW = 2 * EMBED_DIM
SIZES = (32, 64, 128, 128, 96, 64)
MAXC = max(SIZES)
L = 16


def _vsqrt(x):
    i = plsc.bitcast(x, jnp.int32)
    i = jnp.int32(0x5F3759DF) - (i >> 1)
    y = plsc.bitcast(i, jnp.float32)
    xy = x * y
    y = y * (1.5 - 0.5 * xy * y)
    xy = x * y
    y = y * (1.5 - 0.5 * xy * y)
    return x * y


def _trig_body(r_ref, cs_ref):
    r = r_ref[...]
    cs_ref[...] = jnp.concatenate([jnp.cos(r), jnp.sin(r)], axis=1)


def _make_sc_kernel(batch, num_workers, num_rel):
    bpw = batch // num_workers
    assert sum(SIZES) == bpw
    offs = []
    o = 0
    for sz in SIZES:
        offs.append(o)
        o += sz
    n_chunks = len(SIZES)
    mesh = plsc.VectorSubcoreMesh(core_axis_name="c", subcore_axis_name="s")
    nc = plsc.get_sparse_core_info().num_cores

    @functools.partial(
        pl.kernel,
        mesh=mesh,
        out_type=jax.ShapeDtypeStruct((batch,), jnp.float32),
        scratch_types=[
            pltpu.VMEM((2, MAXC), jnp.int32),
            pltpu.VMEM((2, MAXC), jnp.int32),
            pltpu.VMEM((2, MAXC), jnp.int32),
            pltpu.VMEM((2, MAXC, ROW), jnp.float32),
            pltpu.VMEM((2, MAXC, ROW), jnp.float32),
            pltpu.VMEM((2, MAXC, ROW), jnp.float32),
            pltpu.VMEM((bpw,), jnp.float32),
            pltpu.SemaphoreType.DMA,
            pltpu.SemaphoreType.DMA,
        ],
        compiler_params=pltpu.CompilerParams(needs_layout_passes=False),
    )
    def sc_kernel(heads_hbm, rels_hbm, tails_hbm, ent_hbm, cs_hbm,
                  out_hbm, hidx, ridx, tidx, hrows, trows, csrows,
                  outv, sem0, sem1):
        wid = lax.axis_index("s") * nc + lax.axis_index("c")
        sems = (sem0, sem1)
        lanes = lax.iota(jnp.int32, L)
        base0 = wid * bpw

        def stage(g):
            b = g % 2
            sz = SIZES[g]
            sl = pl.ds(base0 + offs[g], sz)
            dst = pl.ds(0, sz)
            pltpu.sync_copy(heads_hbm.at[sl], hidx.at[b, dst])
            pltpu.sync_copy(rels_hbm.at[sl], ridx.at[b, dst])
            pltpu.sync_copy(tails_hbm.at[sl], tidx.at[b, dst])

        def fire(g):
            b = g % 2
            sz = SIZES[g]
            sem = sems[b]
            isl = pl.ds(0, sz)
            return (
                pltpu.async_copy(ent_hbm.at[hidx.at[b, isl]],
                                 hrows.at[b, isl], sem),
                pltpu.async_copy(ent_hbm.at[tidx.at[b, isl]],
                                 trows.at[b, isl], sem),
                pltpu.async_copy(cs_hbm.at[ridx.at[b, isl]],
                                 csrows.at[b, isl], sem),
            )

        stage(0)
        pending = [fire(0)]
        for g in range(n_chunks):
            b = g % 2
            if g + 1 < n_chunks:
                stage(g + 1)
                pending.append(fire(g + 1))
            for cp in pending.pop(0):
                cp.wait()
            hb, tb, cb = hrows.at[b], trows.at[b], csrows.at[b]
            off = offs[g]

            def grp(j, _, off=off, hb=hb, tb=tb, cb=cb):
                base = j * L
                scorev = jnp.zeros((L,), jnp.float32)
                for k in range(L):
                    i = base + k
                    acc = jnp.zeros((L,), jnp.float32)
                    for q in range(EMBED_DIM // L):
                        re = pl.ds(q * L, L)
                        im = pl.ds(EMBED_DIM + q * L, L)
                        h_re = hb[i, re]
                        h_im = hb[i, im]
                        t_re = tb[i, re]
                        t_im = tb[i, im]
                        c = cb[i, re]
                        s = cb[i, im]
                        d_re = h_re * c - h_im * s - t_re
                        d_im = h_re * s + h_im * c - t_im
                        acc = acc + _vsqrt(d_re * d_re + d_im * d_im)
                    scorev = jnp.where(lanes == k, jnp.sum(acc), scorev)
                outv[pl.ds(off + base, L)] = scorev
                return 0

            lax.fori_loop(0, SIZES[g] // L, grp, 0)

        pltpu.sync_copy(outv, out_hbm.at[pl.ds(base0, bpw)])

    return sc_kernel


def kernel(heads, relations, tails, entity_emb, relation_emb):
    batch = heads.shape[0]
    num_rel = relation_emb.shape[0]
    info = plsc.get_sparse_core_info()
    num_workers = info.num_cores * info.num_subcores

    cs_t = pl.pallas_call(
        _trig_body,
        out_shape=jax.ShapeDtypeStruct((num_rel, ROW), jnp.float32),
    )(relation_emb)

    sc = _make_sc_kernel(batch, num_workers, num_rel)
    return sc(heads.astype(jnp.int32), relations.astype(jnp.int32),
              tails.astype(jnp.int32), entity_emb, cs_t)


# speedup vs baseline: 1.1587x; 1.1587x over previous
import functools

import jax
import jax.numpy as jnp
from jax import lax
from jax.experimental import pallas as pl
from jax.experimental.pallas import tpu as pltpu
from jax.experimental.pallas import tpu_sc as plsc

EMBED_DIM = 64
ROW = 2 * EMBED_DIM
CHUNK = 128
NBUF = 2
L = 16


def _vsqrt(x):
    i = plsc.bitcast(x, jnp.int32)
    i = jnp.int32(0x5F3759DF) - (i >> 1)
    y = plsc.bitcast(i, jnp.float32)
    xy = x * y
    y = y * (1.5 - 0.5 * xy * y)
    xy = x * y
    y = y * (1.5 - 0.5 * xy * y)
    return x * y


def _trig_body(r_ref, cs_ref):
    r = r_ref[...]
    cs_ref[...] = jnp.concatenate([jnp.cos(r), jnp.sin(r)], axis=1)


def _make_sc_kernel(batch, num_workers, num_rel):
    n_chunks = batch // (num_workers * CHUNK)
    bpw = batch // num_workers
    mesh = plsc.VectorSubcoreMesh(core_axis_name="c", subcore_axis_name="s")
    nc = plsc.get_sparse_core_info().num_cores

    @functools.partial(
        pl.kernel,
        mesh=mesh,
        out_type=jax.ShapeDtypeStruct((batch,), jnp.float32),
        scratch_types=[
            pltpu.VMEM((n_chunks, CHUNK), jnp.int32),
            pltpu.VMEM((n_chunks, CHUNK), jnp.int32),
            pltpu.VMEM((n_chunks, CHUNK), jnp.int32),
            pltpu.VMEM((NBUF, CHUNK, ROW), jnp.float32),
            pltpu.VMEM((NBUF, CHUNK, ROW), jnp.float32),
            pltpu.VMEM((NBUF, CHUNK, ROW), jnp.float32),
            pltpu.VMEM((bpw,), jnp.float32),
        ] + [pltpu.SemaphoreType.DMA] * NBUF,
        compiler_params=pltpu.CompilerParams(needs_layout_passes=False),
    )
    def sc_kernel(heads_hbm, rels_hbm, tails_hbm, ent_hbm, cs_hbm,
                  out_hbm, hidx, ridx, tidx, hrows, trows, csrows,
                  outv, *sems):
        wid = lax.axis_index("s") * nc + lax.axis_index("c")
        lanes = lax.iota(jnp.int32, L)

        def fire(g):
            b = g % NBUF
            sem = sems[b]
            return (
                pltpu.async_copy(ent_hbm.at[hidx.at[g]], hrows.at[b], sem),
                pltpu.async_copy(ent_hbm.at[tidx.at[g]], trows.at[b], sem),
                pltpu.async_copy(cs_hbm.at[ridx.at[g]], csrows.at[b], sem),
            )

        sl = pl.ds(wid * n_chunks, n_chunks)
        pltpu.sync_copy(heads_hbm.at[sl], hidx)
        cp0h = pltpu.async_copy(ent_hbm.at[hidx.at[0]], hrows.at[0], sems[0])
        pltpu.sync_copy(tails_hbm.at[sl], tidx)
        cp0t = pltpu.async_copy(ent_hbm.at[tidx.at[0]], trows.at[0], sems[0])
        pltpu.sync_copy(rels_hbm.at[sl], ridx)
        cp0c = pltpu.async_copy(cs_hbm.at[ridx.at[0]], csrows.at[0], sems[0])

        pending = [(cp0h, cp0t, cp0c)]
        for g in range(n_chunks):
            b = g % NBUF
            if g + NBUF - 1 < n_chunks:
                pending.append(fire(g + NBUF - 1))
            for cp in pending.pop(0):
                cp.wait()
            hb, tb, cb = hrows.at[b], trows.at[b], csrows.at[b]

            def grp(j, _, g=g, hb=hb, tb=tb, cb=cb):
                base = j * L
                scorev = jnp.zeros((L,), jnp.float32)
                for k in range(L):
                    i = base + k
                    acc = jnp.zeros((L,), jnp.float32)
                    for q in range(EMBED_DIM // L):
                        re = pl.ds(q * L, L)
                        im = pl.ds(EMBED_DIM + q * L, L)
                        h_re = hb[i, re]
                        h_im = hb[i, im]
                        t_re = tb[i, re]
                        t_im = tb[i, im]
                        c = cb[i, re]
                        s = cb[i, im]
                        d_re = h_re * c - h_im * s - t_re
                        d_im = h_re * s + h_im * c - t_im
                        acc = acc + _vsqrt(d_re * d_re + d_im * d_im)
                    scorev = jnp.where(lanes == k, jnp.sum(acc), scorev)
                outv[pl.ds(g * CHUNK + base, L)] = scorev
                return 0

            lax.fori_loop(0, CHUNK // L, grp, 0)

        pltpu.sync_copy(outv, out_hbm.at[pl.ds(wid * bpw, bpw)])

    return sc_kernel


def kernel(heads, relations, tails, entity_emb, relation_emb):
    batch = heads.shape[0]
    num_rel = relation_emb.shape[0]
    info = plsc.get_sparse_core_info()
    num_workers = info.num_cores * info.num_subcores

    cs_t = pl.pallas_call(
        _trig_body,
        out_shape=jax.ShapeDtypeStruct((num_rel, ROW), jnp.float32),
    )(relation_emb)

    n_rows = batch // CHUNK
    heads2 = heads.astype(jnp.int32).reshape(n_rows, CHUNK)
    rels2 = relations.astype(jnp.int32).reshape(n_rows, CHUNK)
    tails2 = tails.astype(jnp.int32).reshape(n_rows, CHUNK)

    sc = _make_sc_kernel(batch, num_workers, num_rel)
    return sc(heads2, rels2, tails2, entity_emb, cs_t)
